# baseline (speedup 1.0000x reference)
import functools

import jax
import jax.numpy as jnp
from jax import lax
from jax.experimental import pallas as pl
from jax.experimental.pallas import tpu as pltpu
from jax.experimental.pallas import tpu_sc as plsc

_N_MESH = 65536
_N_RECV = 1024
_LANES = 128
_SUB = 8
_TILE = _SUB * _LANES
_N_TILES = _N_MESH // _TILE
_ROWS = _N_MESH // _LANES


def _argmin_body(xs_ref, ys_ref, rx_ref, ry_ref, out_ref, cp_ref):
    k = pl.program_id(0)
    rx = rx_ref[k]
    ry = ry_ref[k]
    pos = (lax.broadcasted_iota(jnp.int32, (_SUB, _LANES), 0) * _LANES
           + lax.broadcasted_iota(jnp.int32, (_SUB, _LANES), 1))

    def body(j, carry):
        runmin, runblk = carry
        xb = xs_ref[pl.ds(j * _SUB, _SUB), :]
        yb = ys_ref[pl.ds(j * _SUB, _SUB), :]
        dx = xb - rx
        dy = yb - ry
        d2 = dx * dx + dy * dy
        lt = d2 < runmin
        runmin = jnp.where(lt, d2, runmin)
        runblk = jnp.where(lt, j, runblk)
        return runmin, runblk

    init = (jnp.full((_SUB, _LANES), jnp.inf, jnp.float32),
            jnp.zeros((_SUB, _LANES), jnp.int32))
    runmin, runblk = lax.fori_loop(0, _N_TILES, body, init, unroll=4)
    m = jnp.min(runmin)
    fullidx = runblk * _TILE + pos
    cand = jnp.where(runmin == m, fullidx, jnp.int32(2**30))
    idx = jnp.min(cand)
    out_ref[k] = idx
    r = idx >> 7
    c = idx & 127
    lane = lax.broadcasted_iota(jnp.int32, (1, _LANES), 1)
    sel = lane == c
    xrow = xs_ref[pl.ds(r, 1), :]
    yrow = ys_ref[pl.ds(r, 1), :]
    cp_ref[k, 0] = jnp.sum(jnp.where(sel, xrow, 0.0))
    cp_ref[k, 1] = jnp.sum(jnp.where(sel, yrow, 0.0))


def _argmin_tc(xs, ys, rx, ry):
    return pl.pallas_call(
        _argmin_body,
        grid=(_N_RECV,),
        in_specs=[
            pl.BlockSpec(memory_space=pltpu.VMEM),
            pl.BlockSpec(memory_space=pltpu.VMEM),
            pl.BlockSpec(memory_space=pltpu.SMEM),
            pl.BlockSpec(memory_space=pltpu.SMEM),
        ],
        out_specs=[pl.BlockSpec(memory_space=pltpu.SMEM),
                   pl.BlockSpec(memory_space=pltpu.SMEM)],
        out_shape=[jax.ShapeDtypeStruct((_N_RECV,), jnp.int32),
                   jax.ShapeDtypeStruct((_N_RECV, 2), jnp.float32)],
        compiler_params=pltpu.CompilerParams(
            dimension_semantics=("arbitrary",)),
    )(xs, ys, rx, ry)


_NC = 2
_NS = 16
_NW = _NC * _NS
_L = 16
_ROWS_W = _N_MESH // _NW
_RECV_W = _N_RECV // _NW


def _sc_assemble_body(meshf_hbm, idx_hbm, out3_hbm,
                      idx_v, mesh_v, out3_v):
    w = lax.axis_index("s") * _NC + lax.axis_index("c")
    base = w * _ROWS_W

    pltpu.sync_copy(idx_hbm, idx_v)
    pltpu.sync_copy(meshf_hbm.at[pl.ds(base * 2, _ROWS_W * 2)], mesh_v)

    lane = lax.iota(jnp.int32, _L)
    one_f = jnp.full((_L,), 1.0, jnp.float32)
    zero_f = jnp.zeros((_L,), jnp.float32)
    xymap = (lane >> 1) * 3 + (lane & 1)
    zmap = lane * 3 + 2

    def interleave(t, _):
        v0 = mesh_v[pl.ds(t * 32, _L)]
        v1 = mesh_v[pl.ds(t * 32 + _L, _L)]
        plsc.store_scatter(out3_v, [t * 48 + xymap], v0)
        plsc.store_scatter(out3_v, [t * 48 + 24 + xymap], v1)
        plsc.store_scatter(out3_v, [t * 48 + zmap], zero_f)
        return 0

    lax.fori_loop(0, _ROWS_W * 2 // 32, interleave, 0)

    def scatter_ones(i, _):
        v = idx_v[pl.ds(i * _L, _L)]
        local = v * 3 - (base * 3 - 2)
        msk = (v >= base) & (v < base + _ROWS_W)
        plsc.store_scatter(out3_v, [local], one_f, mask=msk)
        return 0

    lax.fori_loop(0, _N_RECV // _L, scatter_ones, 0)

    @pl.when(w == 0)
    def _():
        plsc.store_scatter(out3_v, [zmap], one_f, mask=lane == 0)

    pltpu.sync_copy(out3_v, out3_hbm.at[pl.ds(base * 3, _ROWS_W * 3)])


@functools.cache
def _sc_assemble_kernel():
    return pl.kernel(
        _sc_assemble_body,
        mesh=plsc.VectorSubcoreMesh(core_axis_name="c", subcore_axis_name="s"),
        out_type=jax.ShapeDtypeStruct((_N_MESH * 3,), jnp.float32),
        scratch_types=[
            pltpu.VMEM((_N_RECV,), jnp.int32),
            pltpu.VMEM((_ROWS_W * 2,), jnp.float32),
            pltpu.VMEM((_ROWS_W * 3,), jnp.float32),
        ],
        compiler_params=pltpu.CompilerParams(
            needs_layout_passes=False, use_tc_tiling_on_sc=False),
    )


def kernel(mesh_2D, receiver_pos):
    xs = mesh_2D[:, 0].reshape(_ROWS, _LANES)
    ys = mesh_2D[:, 1].reshape(_ROWS, _LANES)
    rx = receiver_pos[:, 0]
    ry = receiver_pos[:, 1]
    min_index, closest_points = _argmin_tc(xs, ys, rx, ry)
    out3 = _sc_assemble_kernel()(mesh_2D.reshape(-1), min_index)
    return (out3.reshape(_N_MESH, 3), closest_points, min_index)
